# baseline (speedup 1.0000x reference)
import jax
import jax.numpy as jnp
from jax.experimental import pallas as pl
from jax.experimental.pallas import tpu as pltpu

_ROWS = 8
_W = 4096

_K1 = 42
_KS2 = 42 ^ 0x1BD11BDA
_KS = (0, _K1, _KS2)
_ROT = ((13, 15, 26, 6), (17, 29, 16, 24))
_TINY = 1.1754943508222875e-38
_NEG_INF = float("-inf")
_BIG = 0x7FFFFFFF


def _gumbel_from_x1(x1):
    x0 = x1
    first = True
    for r in range(5):
        for rot in _ROT[r % 2]:
            if first:
                first = False
            else:
                x0 = x0 + x1
            x1 = (x1 << rot) | (x1 >> (32 - rot))
            x1 = x1 ^ x0
        if _KS[(r + 1) % 3]:
            x0 = x0 + jnp.uint32(_KS[(r + 1) % 3])
        x1 = x1 + jnp.uint32((_KS[(r + 2) % 3] + r + 1) & 0xFFFFFFFF)
    bits = x0 ^ x1

    fb = (bits >> 9) | jnp.uint32(0x3F800000)
    f = jax.lax.bitcast_convert_type(fb, jnp.float32) - jnp.float32(1.0)
    u = jnp.maximum(jnp.float32(_TINY), f + jnp.float32(_TINY))
    return -jnp.log(-jnp.log(u))


def _sample_kernel(x_ref, out_ref):
    pid = pl.program_id(0)
    rows, c = x_ref.shape
    n_main = c // _W
    tail = c - n_main * _W

    row_off = (jax.lax.broadcasted_iota(jnp.uint32, (rows, 1), 0)
               + pid.astype(jnp.uint32) * jnp.uint32(rows)) * jnp.uint32(c) + jnp.uint32(_K1)
    lane = jax.lax.broadcasted_iota(jnp.uint32, (rows, _W), 1)

    def body(k, carry):
        accmax, accidx = carry
        base = k * _W
        gum = _gumbel_from_x1(lane + (row_off + base.astype(jnp.uint32)))
        v = x_ref[:, pl.ds(base, _W)] + gum
        upd = accmax < v
        accmax = jnp.where(upd, v, accmax)
        accidx = jnp.where(upd, k, accidx)
        return accmax, accidx

    acc0 = jnp.full((rows, _W), _NEG_INF, jnp.float32)
    idx0 = jnp.zeros((rows, _W), jnp.int32)
    accmax, accidx = jax.lax.fori_loop(0, n_main, body, (acc0, idx0))

    col = accidx * _W + jax.lax.broadcasted_iota(jnp.int32, (rows, _W), 1)
    rowmax = jnp.max(accmax, axis=1, keepdims=True)
    cand = jnp.min(jnp.where(accmax == rowmax, col, _BIG), axis=1)

    if tail:
        lane_t = jax.lax.broadcasted_iota(jnp.uint32, (rows, tail), 1)
        base_t = n_main * _W
        gum_t = _gumbel_from_x1(lane_t + (row_off + jnp.uint32(base_t)))
        v_t = x_ref[:, base_t:c] + gum_t
        col_t = lane_t.astype(jnp.int32) + base_t
        rowmax_t = jnp.max(v_t, axis=1, keepdims=True)
        cand_t = jnp.min(jnp.where(v_t == rowmax_t, col_t, _BIG), axis=1)
        better_t = rowmax_t[:, 0] > rowmax[:, 0]
        tok = jnp.where(better_t, cand_t, cand)
        tok = jnp.where(rowmax_t[:, 0] == rowmax[:, 0], jnp.minimum(cand, cand_t), tok)
    else:
        tok = cand

    out_ref[0, 0, :] = tok


def kernel(logits):
    b, n, c = logits.shape
    x2 = logits.reshape(b * n, c)
    grid = (b * n) // _ROWS
    out = pl.pallas_call(
        _sample_kernel,
        grid=(grid,),
        in_specs=[pl.BlockSpec((_ROWS, c), lambda i: (i, 0))],
        out_specs=pl.BlockSpec((1, 1, _ROWS), lambda i: (i, 0, 0)),
        out_shape=jax.ShapeDtypeStruct((grid, 1, _ROWS), jnp.int32),
        compiler_params=pltpu.CompilerParams(
            dimension_semantics=("arbitrary",),
        ),
    )(x2)
    return out.reshape(b, n)


# speedup vs baseline: 1.0950x; 1.0950x over previous
import jax
import jax.numpy as jnp
from jax.experimental import pallas as pl
from jax.experimental.pallas import tpu as pltpu

_ROWS = 8
_W = 2048

_K1 = 42
_KS2 = 42 ^ 0x1BD11BDA
_KS = (0, _K1, _KS2)
_ROT = ((13, 15, 26, 6), (17, 29, 16, 24))
_TINY = 1.1754943508222875e-38
_NEG_INF = float("-inf")
_BIG = 0x7FFFFFFF


def _gumbel_from_x1(x1):
    x0 = x1
    first = True
    for r in range(5):
        for rot in _ROT[r % 2]:
            if first:
                first = False
            else:
                x0 = x0 + x1
            x1 = (x1 << rot) | (x1 >> (32 - rot))
            x1 = x1 ^ x0
        if _KS[(r + 1) % 3]:
            x0 = x0 + jnp.uint32(_KS[(r + 1) % 3])
        x1 = x1 + jnp.uint32((_KS[(r + 2) % 3] + r + 1) & 0xFFFFFFFF)
    bits = x0 ^ x1

    fb = (bits >> 9) | jnp.uint32(0x3F800000)
    f = jax.lax.bitcast_convert_type(fb, jnp.float32) - jnp.float32(1.0)
    u = jnp.maximum(f, jnp.float32(_TINY))
    return -jnp.log(-jnp.log(u))


def _sample_kernel(x_ref, out_ref):
    pid = pl.program_id(0)
    rows, c = x_ref.shape
    n_main = c // _W
    tail = c - n_main * _W

    row_off = (jax.lax.broadcasted_iota(jnp.uint32, (rows, 1), 0)
               + pid.astype(jnp.uint32) * jnp.uint32(rows)) * jnp.uint32(c) + jnp.uint32(_K1)
    lane = jax.lax.broadcasted_iota(jnp.uint32, (rows, _W), 1)

    def body(k, carry):
        accmax, accidx = carry
        base = k * _W
        gum = _gumbel_from_x1(lane + (row_off + base.astype(jnp.uint32)))
        v = x_ref[:, pl.ds(base, _W)] + gum
        upd = accmax < v
        accmax = jnp.where(upd, v, accmax)
        accidx = jnp.where(upd, k, accidx)
        return accmax, accidx

    acc0 = jnp.full((rows, _W), _NEG_INF, jnp.float32)
    idx0 = jnp.zeros((rows, _W), jnp.int32)
    accmax, accidx = jax.lax.fori_loop(0, n_main, body, (acc0, idx0))

    col = accidx * _W + jax.lax.broadcasted_iota(jnp.int32, (rows, _W), 1)
    rowmax = jnp.max(accmax, axis=1, keepdims=True)
    cand = jnp.min(jnp.where(accmax == rowmax, col, _BIG), axis=1)

    if tail:
        lane_t = jax.lax.broadcasted_iota(jnp.uint32, (rows, tail), 1)
        base_t = n_main * _W
        gum_t = _gumbel_from_x1(lane_t + (row_off + jnp.uint32(base_t)))
        v_t = x_ref[:, base_t:c] + gum_t
        col_t = lane_t.astype(jnp.int32) + base_t
        rowmax_t = jnp.max(v_t, axis=1, keepdims=True)
        cand_t = jnp.min(jnp.where(v_t == rowmax_t, col_t, _BIG), axis=1)
        better_t = rowmax_t[:, 0] > rowmax[:, 0]
        tok = jnp.where(better_t, cand_t, cand)
        tok = jnp.where(rowmax_t[:, 0] == rowmax[:, 0], jnp.minimum(cand, cand_t), tok)
    else:
        tok = cand

    out_ref[0, 0, :] = tok


def kernel(logits):
    b, n, c = logits.shape
    x2 = logits.reshape(b * n, c)
    grid = (b * n) // _ROWS
    out = pl.pallas_call(
        _sample_kernel,
        grid=(grid,),
        in_specs=[pl.BlockSpec((_ROWS, c), lambda i: (i, 0))],
        out_specs=pl.BlockSpec((1, 1, _ROWS), lambda i: (i, 0, 0)),
        out_shape=jax.ShapeDtypeStruct((grid, 1, _ROWS), jnp.int32),
        compiler_params=pltpu.CompilerParams(
            dimension_semantics=("arbitrary",),
        ),
    )(x2)
    return out.reshape(b, n)


# speedup vs baseline: 1.1601x; 1.0594x over previous
import jax
import jax.numpy as jnp
from jax.experimental import pallas as pl
from jax.experimental.pallas import tpu as pltpu

_ROWS = 8
_W = 1024

_K1 = 42
_KS2 = 42 ^ 0x1BD11BDA
_KS = (0, _K1, _KS2)
_ROT = ((13, 15, 26, 6), (17, 29, 16, 24))
_TINY = 1.1754943508222875e-38
_NEG_INF = float("-inf")
_BIG = 0x7FFFFFFF


def _gumbel_from_x1(x1):
    x0 = x1
    first = True
    for r in range(5):
        for rot in _ROT[r % 2]:
            if first:
                first = False
            else:
                x0 = x0 + x1
            x1 = (x1 << rot) | (x1 >> (32 - rot))
            x1 = x1 ^ x0
        if _KS[(r + 1) % 3]:
            x0 = x0 + jnp.uint32(_KS[(r + 1) % 3])
        x1 = x1 + jnp.uint32((_KS[(r + 2) % 3] + r + 1) & 0xFFFFFFFF)
    bits = x0 ^ x1

    fb = (bits >> 9) | jnp.uint32(0x3F800000)
    f = jax.lax.bitcast_convert_type(fb, jnp.float32) - jnp.float32(1.0)
    u = jnp.maximum(f, jnp.float32(_TINY))
    return -jnp.log(-jnp.log(u))


def _sample_kernel(x_ref, out_ref):
    pid = pl.program_id(0)
    rows, c = x_ref.shape
    n_main = c // _W
    tail = c - n_main * _W

    row_off = (jax.lax.broadcasted_iota(jnp.uint32, (rows, 1), 0)
               + pid.astype(jnp.uint32) * jnp.uint32(rows)) * jnp.uint32(c) + jnp.uint32(_K1)
    lane = jax.lax.broadcasted_iota(jnp.uint32, (rows, _W), 1)

    def update(accmax, accidx, v, k):
        upd = accmax < v
        return jnp.where(upd, v, accmax), jnp.where(upd, k, accidx)

    def body(k, carry):
        accmax, accidx = carry
        k2 = 2 * k
        base = k2 * _W
        x1a = lane + (row_off + base.astype(jnp.uint32))
        x1b = x1a + jnp.uint32(_W)
        va = x_ref[:, pl.ds(base, _W)] + _gumbel_from_x1(x1a)
        vb = x_ref[:, pl.ds(base + _W, _W)] + _gumbel_from_x1(x1b)
        accmax, accidx = update(accmax, accidx, va, k2)
        accmax, accidx = update(accmax, accidx, vb, k2 + 1)
        return accmax, accidx

    acc0 = jnp.full((rows, _W), _NEG_INF, jnp.float32)
    idx0 = jnp.zeros((rows, _W), jnp.int32)
    accmax, accidx = jax.lax.fori_loop(0, n_main // 2, body, (acc0, idx0))
    if n_main % 2:
        base = (n_main - 1) * _W
        v = x_ref[:, pl.ds(base, _W)] + _gumbel_from_x1(
            lane + (row_off + jnp.uint32(base)))
        accmax, accidx = update(accmax, accidx, v, n_main - 1)

    col = accidx * _W + jax.lax.broadcasted_iota(jnp.int32, (rows, _W), 1)
    rowmax = jnp.max(accmax, axis=1, keepdims=True)
    cand = jnp.min(jnp.where(accmax == rowmax, col, _BIG), axis=1)

    if tail:
        lane_t = jax.lax.broadcasted_iota(jnp.uint32, (rows, tail), 1)
        base_t = n_main * _W
        gum_t = _gumbel_from_x1(lane_t + (row_off + jnp.uint32(base_t)))
        v_t = x_ref[:, base_t:c] + gum_t
        col_t = lane_t.astype(jnp.int32) + base_t
        rowmax_t = jnp.max(v_t, axis=1, keepdims=True)
        cand_t = jnp.min(jnp.where(v_t == rowmax_t, col_t, _BIG), axis=1)
        better_t = rowmax_t[:, 0] > rowmax[:, 0]
        tok = jnp.where(better_t, cand_t, cand)
        tok = jnp.where(rowmax_t[:, 0] == rowmax[:, 0], jnp.minimum(cand, cand_t), tok)
    else:
        tok = cand

    out_ref[0, 0, :] = tok


def kernel(logits):
    b, n, c = logits.shape
    x2 = logits.reshape(b * n, c)
    grid = (b * n) // _ROWS
    out = pl.pallas_call(
        _sample_kernel,
        grid=(grid,),
        in_specs=[pl.BlockSpec((_ROWS, c), lambda i: (i, 0))],
        out_specs=pl.BlockSpec((1, 1, _ROWS), lambda i: (i, 0, 0)),
        out_shape=jax.ShapeDtypeStruct((grid, 1, _ROWS), jnp.int32),
        compiler_params=pltpu.CompilerParams(
            dimension_semantics=("arbitrary",),
        ),
    )(x2)
    return out.reshape(b, n)


# speedup vs baseline: 1.1977x; 1.0325x over previous
import jax
import jax.numpy as jnp
from jax.experimental import pallas as pl
from jax.experimental.pallas import tpu as pltpu

_ROWS = 8
_W = 1024

_K1 = 42
_KS2 = 42 ^ 0x1BD11BDA
_KS = (0, _K1, _KS2)
_ROT = ((13, 15, 26, 6), (17, 29, 16, 24))
_TINY = 1.1754943508222875e-38
_NEG_INF = float("-inf")
_BIG = 0x7FFFFFFF


def _gumbel_from_x1(x1):
    x0 = x1
    first = True
    for r in range(5):
        for rot in _ROT[r % 2]:
            if first:
                first = False
            else:
                x0 = x0 + x1
            x1 = (x1 << rot) | (x1 >> (32 - rot))
            x1 = x1 ^ x0
        if _KS[(r + 1) % 3]:
            x0 = x0 + jnp.uint32(_KS[(r + 1) % 3])
        x1 = x1 + jnp.uint32((_KS[(r + 2) % 3] + r + 1) & 0xFFFFFFFF)
    bits = x0 ^ x1

    fb = (bits >> 9) | jnp.uint32(0x3F800000)
    f = jax.lax.bitcast_convert_type(fb, jnp.float32) - jnp.float32(1.0)
    u = jnp.maximum(f, jnp.float32(_TINY))
    return -jnp.log(-jnp.log(u))


def _sample_kernel(x_ref, out_ref):
    pid = pl.program_id(0)
    rows, c = x_ref.shape
    n_main = c // _W
    tail = c - n_main * _W

    row_off = (jax.lax.broadcasted_iota(jnp.uint32, (rows, 1), 0)
               + pid.astype(jnp.uint32) * jnp.uint32(rows)) * jnp.uint32(c) + jnp.uint32(_K1)
    lane = jax.lax.broadcasted_iota(jnp.uint32, (rows, _W), 1)

    def update(accmax, accidx, v, k):
        upd = accmax < v
        return jnp.where(upd, v, accmax), jnp.where(upd, k, accidx)

    _UNROLL = 4

    def body(k, carry):
        accmax, accidx = carry
        k0 = _UNROLL * k
        base = k0 * _W
        x1a = lane + (row_off + base.astype(jnp.uint32))
        vs = []
        for j in range(_UNROLL):
            vs.append(x_ref[:, pl.ds(base + j * _W, _W)]
                      + _gumbel_from_x1(x1a + jnp.uint32(j * _W)))
        for j in range(_UNROLL):
            accmax, accidx = update(accmax, accidx, vs[j], k0 + j)
        return accmax, accidx

    acc0 = jnp.full((rows, _W), _NEG_INF, jnp.float32)
    idx0 = jnp.zeros((rows, _W), jnp.int32)
    accmax, accidx = jax.lax.fori_loop(0, n_main // _UNROLL, body, (acc0, idx0))
    for kk in range((n_main // _UNROLL) * _UNROLL, n_main):
        base = kk * _W
        v = x_ref[:, pl.ds(base, _W)] + _gumbel_from_x1(
            lane + (row_off + jnp.uint32(base)))
        accmax, accidx = update(accmax, accidx, v, kk)

    col = accidx * _W + jax.lax.broadcasted_iota(jnp.int32, (rows, _W), 1)
    rowmax = jnp.max(accmax, axis=1, keepdims=True)
    cand = jnp.min(jnp.where(accmax == rowmax, col, _BIG), axis=1)

    if tail:
        lane_t = jax.lax.broadcasted_iota(jnp.uint32, (rows, tail), 1)
        base_t = n_main * _W
        gum_t = _gumbel_from_x1(lane_t + (row_off + jnp.uint32(base_t)))
        v_t = x_ref[:, base_t:c] + gum_t
        col_t = lane_t.astype(jnp.int32) + base_t
        rowmax_t = jnp.max(v_t, axis=1, keepdims=True)
        cand_t = jnp.min(jnp.where(v_t == rowmax_t, col_t, _BIG), axis=1)
        better_t = rowmax_t[:, 0] > rowmax[:, 0]
        tok = jnp.where(better_t, cand_t, cand)
        tok = jnp.where(rowmax_t[:, 0] == rowmax[:, 0], jnp.minimum(cand, cand_t), tok)
    else:
        tok = cand

    out_ref[0, 0, :] = tok


def kernel(logits):
    b, n, c = logits.shape
    x2 = logits.reshape(b * n, c)
    grid = (b * n) // _ROWS
    out = pl.pallas_call(
        _sample_kernel,
        grid=(grid,),
        in_specs=[pl.BlockSpec((_ROWS, c), lambda i: (i, 0))],
        out_specs=pl.BlockSpec((1, 1, _ROWS), lambda i: (i, 0, 0)),
        out_shape=jax.ShapeDtypeStruct((grid, 1, _ROWS), jnp.int32),
        compiler_params=pltpu.CompilerParams(
            dimension_semantics=("arbitrary",),
        ),
    )(x2)
    return out.reshape(b, n)
